# baseline (speedup 1.0000x reference)
import functools

import jax
import jax.numpy as jnp
from jax.experimental import pallas as pl
from jax.experimental.pallas import tpu as pltpu

_INTERPRET = False


def _pass1_body(cap, B, E, x_ref, wgT_ref, L_ref,
                g1_ref, g2_ref, e1_ref, e2_ref, loc1_ref, rank2_ref,
                me_ref, cnt1_ref, carry2_ref):
    i = pl.program_id(0)

    @pl.when(i == 0)
    def _init():
        me_ref[...] = jnp.zeros_like(me_ref)
        cnt1_ref[...] = jnp.zeros_like(cnt1_ref)
        carry2_ref[...] = jnp.zeros_like(carry2_ref)

    logits = jnp.dot(x_ref[...], wgT_ref[...],
                     preferred_element_type=jnp.float32)

    col = jax.lax.broadcasted_iota(jnp.int32, (B, E), 1)
    m1 = jnp.max(logits, axis=1, keepdims=True)
    is1 = logits == m1
    e1 = jnp.min(jnp.where(is1, col, E), axis=1)
    mask1 = col == e1[:, None]
    masked = jnp.where(mask1, -jnp.inf, logits)
    m2 = jnp.max(masked, axis=1, keepdims=True)
    is2 = masked == m2
    e2 = jnp.min(jnp.where(is2, col, E), axis=1)
    mask2 = col == e2[:, None]

    ex = jnp.exp(logits - m1)
    denom = jnp.sum(ex, axis=1, keepdims=True)
    gates = ex / denom
    m1f = mask1.astype(jnp.float32)
    m2f = mask2.astype(jnp.float32)
    g1 = jnp.sum(gates * m1f, axis=1)
    g2 = jnp.sum(gates * m2f, axis=1)
    me_ref[...] += jnp.sum(gates, axis=0, keepdims=True)

    m12 = jnp.concatenate([m1f, m2f], axis=1).astype(jnp.bfloat16)
    C = jnp.dot(L_ref[...], m12, preferred_element_type=jnp.float32)
    C1 = C[:, :E]
    C2 = C[:, E:]
    rank1 = jnp.sum(C1 * m1f, axis=1) - 1.0 \
        + jnp.sum(cnt1_ref[...] * m1f, axis=1)
    rank2 = jnp.sum(C2 * m2f, axis=1) - 1.0 \
        + jnp.sum(carry2_ref[...] * m2f, axis=1)
    cnt1_ref[...] += jnp.sum(m1f, axis=0, keepdims=True)
    carry2_ref[...] += jnp.sum(m2f, axis=0, keepdims=True)

    loc1 = jnp.where(rank1 < cap, rank1, 0.0)

    g1_ref[...] = g1
    g2_ref[...] = g2
    e1_ref[...] = e1
    e2_ref[...] = e2
    loc1_ref[...] = loc1.astype(jnp.int32)
    rank2_ref[...] = rank2.astype(jnp.int32)


def _pass2_body(cap, PB, E, scale, rank2_ref, e2_ref, cnt1_ref, me_ref,
                loc2_ref, laux_ref):
    i = pl.program_id(0)
    e2 = e2_ref[...]
    col = jax.lax.broadcasted_iota(jnp.int32, (PB, E), 1)
    oh = (col == e2[:, None]).astype(jnp.float32)
    cntg = jnp.sum(cnt1_ref[...] * oh, axis=1)
    loc2 = rank2_ref[...] + cntg.astype(jnp.int32)
    loc2_ref[...] = jnp.where(loc2 < cap, loc2, 0)

    @pl.when(i == 0)
    def _():
        cnt1 = cnt1_ref[...]
        ce = jnp.minimum(cnt1, float(cap))
        laux_ref[...] = jnp.sum(me_ref[...] * ce, axis=1,
                                keepdims=True) * scale


def kernel(input, wg):
    N, D = input.shape
    E = wg.shape[0]
    B = 512
    NB = N // B
    cap = 2 * ((N + E - 1) // E)

    wgT = wg.T
    L = jnp.tril(jnp.ones((B, B), jnp.bfloat16))

    outs1 = pl.pallas_call(
        functools.partial(_pass1_body, cap, B, E),
        grid=(NB,),
        in_specs=[
            pl.BlockSpec((B, D), lambda i: (i, 0)),
            pl.BlockSpec((D, E), lambda i: (0, 0)),
            pl.BlockSpec((B, B), lambda i: (0, 0)),
        ],
        out_specs=[
            pl.BlockSpec((B,), lambda i: (i,)),
            pl.BlockSpec((B,), lambda i: (i,)),
            pl.BlockSpec((B,), lambda i: (i,)),
            pl.BlockSpec((B,), lambda i: (i,)),
            pl.BlockSpec((B,), lambda i: (i,)),
            pl.BlockSpec((B,), lambda i: (i,)),
            pl.BlockSpec((1, E), lambda i: (0, 0)),
            pl.BlockSpec((1, E), lambda i: (0, 0)),
        ],
        out_shape=[
            jax.ShapeDtypeStruct((N,), jnp.float32),
            jax.ShapeDtypeStruct((N,), jnp.float32),
            jax.ShapeDtypeStruct((N,), jnp.int32),
            jax.ShapeDtypeStruct((N,), jnp.int32),
            jax.ShapeDtypeStruct((N,), jnp.int32),
            jax.ShapeDtypeStruct((N,), jnp.int32),
            jax.ShapeDtypeStruct((1, E), jnp.float32),
            jax.ShapeDtypeStruct((1, E), jnp.float32),
        ],
        scratch_shapes=[pltpu.VMEM((1, E), jnp.float32)],
        interpret=_INTERPRET,
    )(input, wgT, L)
    g1, g2, e1, e2, loc1, rank2, me, cnt1 = outs1

    PB = 4096
    NB2 = N // PB
    scale = float(E) / (float(N) * float(N))
    loc2, laux = pl.pallas_call(
        functools.partial(_pass2_body, cap, PB, E, scale),
        grid=(NB2,),
        in_specs=[
            pl.BlockSpec((PB,), lambda i: (i,)),
            pl.BlockSpec((PB,), lambda i: (i,)),
            pl.BlockSpec((1, E), lambda i: (0, 0)),
            pl.BlockSpec((1, E), lambda i: (0, 0)),
        ],
        out_specs=[
            pl.BlockSpec((PB,), lambda i: (i,)),
            pl.BlockSpec((1, 1), lambda i: (0, 0)),
        ],
        out_shape=[
            jax.ShapeDtypeStruct((N,), jnp.int32),
            jax.ShapeDtypeStruct((1, 1), jnp.float32),
        ],
        interpret=_INTERPRET,
    )(rank2, e2, cnt1, me)

    return (laux[0, 0], g1, g2, e1, e2, loc1, loc2)


# speedup vs baseline: 3.9872x; 3.9872x over previous
import functools

import jax
import jax.numpy as jnp
from jax import lax
from jax.experimental import pallas as pl
from jax.experimental.pallas import tpu as pltpu

_INTERPRET = False


def _pass1_body(cap, B, E, NB, scale, x_ref, wg_ref, U_ref,
                g1_ref, g2_ref, e1_ref, e2_ref, loc1_ref, rank2_ref,
                cnt1_ref, laux_ref, carry_ref, me_acc_ref):
    i = pl.program_id(0)

    @pl.when(i == 0)
    def _init():
        carry_ref[...] = jnp.zeros_like(carry_ref)
        me_acc_ref[...] = jnp.zeros_like(me_acc_ref)

    logitsT = lax.dot_general(
        wg_ref[...], x_ref[...],
        dimension_numbers=(((1,), (1,)), ((), ())),
        preferred_element_type=jnp.float32)

    row = lax.broadcasted_iota(jnp.int32, (E, B), 0)
    m1 = jnp.max(logitsT, axis=0)
    is1 = logitsT == m1[None, :]
    e1 = jnp.min(jnp.where(is1, row, E), axis=0)
    M1 = row == e1[None, :]
    masked = jnp.where(M1, -jnp.inf, logitsT)
    m2 = jnp.max(masked, axis=0)
    is2 = masked == m2[None, :]
    e2 = jnp.min(jnp.where(is2, row, E), axis=0)
    M2 = row == e2[None, :]

    ex = jnp.exp(logitsT - m1[None, :])
    denom = jnp.sum(ex, axis=0)
    g1 = 1.0 / denom
    g2 = jnp.exp(m2 - m1) * g1
    me_acc_ref[...] += ex * g1[None, :]

    M12 = jnp.concatenate([M1, M2], axis=0).astype(jnp.bfloat16)
    C12 = lax.dot_general(M12, U_ref[...],
                          dimension_numbers=(((1,), (0,)), ((), ())),
                          preferred_element_type=jnp.float32)
    carry = carry_ref[...]
    CC = C12 + carry
    rank1 = jnp.sum(jnp.where(M1, CC[:E], 0.0), axis=0) - 1.0
    rank2 = jnp.sum(jnp.where(M2, CC[E:], 0.0), axis=0) - 1.0
    carry_ref[...] = carry + C12[:, B - 1:B]

    loc1 = jnp.where(rank1 < cap, rank1, 0.0)

    g1_ref[...] = g1
    g2_ref[...] = g2
    e1_ref[...] = e1
    e2_ref[...] = e2
    loc1_ref[...] = loc1.astype(jnp.int32)
    rank2_ref[...] = rank2.astype(jnp.int32)

    @pl.when(i == NB - 1)
    def _fin():
        cnt1 = carry_ref[...][:E]
        cnt1_ref[...] = cnt1
        me = jnp.sum(me_acc_ref[...], axis=1, keepdims=True)
        ce = jnp.minimum(cnt1, float(cap))
        laux_ref[...] = jnp.sum(me * ce, axis=0, keepdims=True) * scale


def _pass2_body(cap, PB, E, rank2_ref, e2_ref, cnt1_ref, loc2_ref):
    e2 = e2_ref[...]
    row = lax.broadcasted_iota(jnp.int32, (E, PB), 0)
    oh = row == e2[None, :]
    cntg = jnp.sum(jnp.where(oh, cnt1_ref[...], 0.0), axis=0)
    loc2 = rank2_ref[...] + cntg.astype(jnp.int32)
    loc2_ref[...] = jnp.where(loc2 < cap, loc2, 0)


def kernel(input, wg):
    N, D = input.shape
    E = wg.shape[0]
    B = 512
    NB = N // B
    cap = 2 * ((N + E - 1) // E)
    scale = float(E) / (float(N) * float(N))

    U = jnp.triu(jnp.ones((B, B), jnp.bfloat16))

    outs1 = pl.pallas_call(
        functools.partial(_pass1_body, cap, B, E, NB, scale),
        grid=(NB,),
        in_specs=[
            pl.BlockSpec((B, D), lambda i: (i, 0)),
            pl.BlockSpec((E, D), lambda i: (0, 0)),
            pl.BlockSpec((B, B), lambda i: (0, 0)),
        ],
        out_specs=[
            pl.BlockSpec((B,), lambda i: (i,)),
            pl.BlockSpec((B,), lambda i: (i,)),
            pl.BlockSpec((B,), lambda i: (i,)),
            pl.BlockSpec((B,), lambda i: (i,)),
            pl.BlockSpec((B,), lambda i: (i,)),
            pl.BlockSpec((B,), lambda i: (i,)),
            pl.BlockSpec((E, 1), lambda i: (0, 0)),
            pl.BlockSpec((1, 1), lambda i: (0, 0)),
        ],
        out_shape=[
            jax.ShapeDtypeStruct((N,), jnp.float32),
            jax.ShapeDtypeStruct((N,), jnp.float32),
            jax.ShapeDtypeStruct((N,), jnp.int32),
            jax.ShapeDtypeStruct((N,), jnp.int32),
            jax.ShapeDtypeStruct((N,), jnp.int32),
            jax.ShapeDtypeStruct((N,), jnp.int32),
            jax.ShapeDtypeStruct((E, 1), jnp.float32),
            jax.ShapeDtypeStruct((1, 1), jnp.float32),
        ],
        scratch_shapes=[
            pltpu.VMEM((2 * E, 1), jnp.float32),
            pltpu.VMEM((E, B), jnp.float32),
        ],
        interpret=_INTERPRET,
    )(input, wg, U)
    g1, g2, e1, e2, loc1, rank2, cnt1, laux = outs1

    PB = 2048
    NB2 = N // PB
    loc2 = pl.pallas_call(
        functools.partial(_pass2_body, cap, PB, E),
        grid=(NB2,),
        in_specs=[
            pl.BlockSpec((PB,), lambda i: (i,)),
            pl.BlockSpec((PB,), lambda i: (i,)),
            pl.BlockSpec((E, 1), lambda i: (0, 0)),
        ],
        out_specs=pl.BlockSpec((PB,), lambda i: (i,)),
        out_shape=jax.ShapeDtypeStruct((N,), jnp.int32),
        interpret=_INTERPRET,
    )(rank2, e2, cnt1)

    return (laux[0, 0], g1, g2, e1, e2, loc1, loc2)


# speedup vs baseline: 5.1648x; 1.2953x over previous
import functools

import jax
import jax.numpy as jnp
from jax import lax
from jax.experimental import pallas as pl
from jax.experimental.pallas import tpu as pltpu

_INTERPRET = False


def _pass1_body(cap, B, E, NB, scale, x_ref, wg_ref, U_ref,
                g1_ref, g2_ref, e1_ref, e2_ref, loc1_ref, rank2_ref,
                cnt1_ref, laux_ref, carry_ref, me_acc_ref):
    i = pl.program_id(0)

    @pl.when(i == 0)
    def _init():
        carry_ref[...] = jnp.zeros_like(carry_ref)
        me_acc_ref[...] = jnp.zeros_like(me_acc_ref)

    logitsT = lax.dot_general(
        wg_ref[...], x_ref[...],
        dimension_numbers=(((1,), (1,)), ((), ())),
        preferred_element_type=jnp.float32)

    row = lax.broadcasted_iota(jnp.int32, (E, B), 0)
    m1 = jnp.max(logitsT, axis=0)
    is1 = logitsT == m1[None, :]
    e1 = jnp.min(jnp.where(is1, row, E), axis=0)
    M1 = row == e1[None, :]
    masked = jnp.where(M1, -jnp.inf, logitsT)
    m2 = jnp.max(masked, axis=0)
    is2 = masked == m2[None, :]
    e2 = jnp.min(jnp.where(is2, row, E), axis=0)
    M2 = row == e2[None, :]

    ex = jnp.exp(logitsT - m1[None, :])
    denom = jnp.sum(ex, axis=0)
    g1 = 1.0 / denom
    g2 = jnp.exp(m2 - m1) * g1
    me_acc_ref[...] += ex * g1[None, :]

    M12 = jnp.concatenate([M1, M2], axis=0).astype(jnp.bfloat16)
    C12 = lax.dot_general(M12, U_ref[...],
                          dimension_numbers=(((1,), (0,)), ((), ())),
                          preferred_element_type=jnp.float32)
    carry = carry_ref[...]
    CC = C12 + carry
    rank1 = jnp.sum(jnp.where(M1, CC[:E], 0.0), axis=0) - 1.0
    rank2 = jnp.sum(jnp.where(M2, CC[E:], 0.0), axis=0) - 1.0
    carry_ref[...] = carry + C12[:, B - 1:B]

    loc1 = jnp.where(rank1 < cap, rank1, 0.0)

    g1_ref[...] = g1
    g2_ref[...] = g2
    e1_ref[...] = e1
    e2_ref[...] = e2
    loc1_ref[...] = loc1.astype(jnp.int32)
    rank2_ref[...] = rank2.astype(jnp.int32)

    @pl.when(i == NB - 1)
    def _fin():
        cnt1 = carry_ref[...][:E]
        cnt1_ref[...] = cnt1
        me = jnp.sum(me_acc_ref[...], axis=1, keepdims=True)
        ce = jnp.minimum(cnt1, float(cap))
        laux_ref[...] = jnp.sum(me * ce, axis=0, keepdims=True) * scale


def _pass2_body(cap, PB, E, rank2_ref, e2_ref, cnt1_ref, loc2_ref):
    e2 = e2_ref[...]
    row = lax.broadcasted_iota(jnp.int32, (E, PB), 0)
    oh = row == e2[None, :]
    cntg = jnp.sum(jnp.where(oh, cnt1_ref[...], 0.0), axis=0)
    loc2 = rank2_ref[...] + cntg.astype(jnp.int32)
    loc2_ref[...] = jnp.where(loc2 < cap, loc2, 0)


def kernel(input, wg):
    N, D = input.shape
    E = wg.shape[0]
    B = 1024
    NB = N // B
    cap = 2 * ((N + E - 1) // E)
    scale = float(E) / (float(N) * float(N))

    U = jnp.triu(jnp.ones((B, B), jnp.bfloat16))

    outs1 = pl.pallas_call(
        functools.partial(_pass1_body, cap, B, E, NB, scale),
        grid=(NB,),
        in_specs=[
            pl.BlockSpec((B, D), lambda i: (i, 0)),
            pl.BlockSpec((E, D), lambda i: (0, 0)),
            pl.BlockSpec((B, B), lambda i: (0, 0)),
        ],
        out_specs=[
            pl.BlockSpec((B,), lambda i: (i,)),
            pl.BlockSpec((B,), lambda i: (i,)),
            pl.BlockSpec((B,), lambda i: (i,)),
            pl.BlockSpec((B,), lambda i: (i,)),
            pl.BlockSpec((B,), lambda i: (i,)),
            pl.BlockSpec((B,), lambda i: (i,)),
            pl.BlockSpec((E, 1), lambda i: (0, 0)),
            pl.BlockSpec((1, 1), lambda i: (0, 0)),
        ],
        out_shape=[
            jax.ShapeDtypeStruct((N,), jnp.float32),
            jax.ShapeDtypeStruct((N,), jnp.float32),
            jax.ShapeDtypeStruct((N,), jnp.int32),
            jax.ShapeDtypeStruct((N,), jnp.int32),
            jax.ShapeDtypeStruct((N,), jnp.int32),
            jax.ShapeDtypeStruct((N,), jnp.int32),
            jax.ShapeDtypeStruct((E, 1), jnp.float32),
            jax.ShapeDtypeStruct((1, 1), jnp.float32),
        ],
        scratch_shapes=[
            pltpu.VMEM((2 * E, 1), jnp.float32),
            pltpu.VMEM((E, B), jnp.float32),
        ],
        interpret=_INTERPRET,
    )(input, wg, U)
    g1, g2, e1, e2, loc1, rank2, cnt1, laux = outs1

    PB = 2048
    NB2 = N // PB
    loc2 = pl.pallas_call(
        functools.partial(_pass2_body, cap, PB, E),
        grid=(NB2,),
        in_specs=[
            pl.BlockSpec((PB,), lambda i: (i,)),
            pl.BlockSpec((PB,), lambda i: (i,)),
            pl.BlockSpec((E, 1), lambda i: (0, 0)),
        ],
        out_specs=pl.BlockSpec((PB,), lambda i: (i,)),
        out_shape=jax.ShapeDtypeStruct((N,), jnp.int32),
        interpret=_INTERPRET,
    )(rank2, e2, cnt1)

    return (laux[0, 0], g1, g2, e1, e2, loc1, loc2)


# speedup vs baseline: 7.8694x; 1.5237x over previous
import functools

import jax
import jax.numpy as jnp
from jax import lax
from jax.experimental import pallas as pl
from jax.experimental.pallas import tpu as pltpu

_INTERPRET = False


def _body(cap, B, E, NB, scale, x_ref, wg_ref, U_ref,
          g1_ref, g2_ref, e1_ref, e2_ref, loc1_ref, loc2_ref,
          cnt1_ref, laux_ref, carry_ref, me_acc_ref, r2_sc, e2_sc):
    i = pl.program_id(0)

    @pl.when(i == 0)
    def _init():
        carry_ref[...] = jnp.zeros_like(carry_ref)
        me_acc_ref[...] = jnp.zeros_like(me_acc_ref)

    @pl.when(i < NB)
    def _phase1():
        logitsT = lax.dot_general(
            wg_ref[...], x_ref[...],
            dimension_numbers=(((1,), (1,)), ((), ())),
            preferred_element_type=jnp.float32)

        row = lax.broadcasted_iota(jnp.int32, (E, B), 0)
        m1 = jnp.max(logitsT, axis=0)
        is1 = logitsT == m1[None, :]
        e1 = jnp.min(jnp.where(is1, row, E), axis=0)
        M1 = row == e1[None, :]
        masked = jnp.where(M1, -jnp.inf, logitsT)
        m2 = jnp.max(masked, axis=0)
        is2 = masked == m2[None, :]
        e2 = jnp.min(jnp.where(is2, row, E), axis=0)
        M2 = row == e2[None, :]

        ex = jnp.exp(logitsT - m1[None, :])
        denom = jnp.sum(ex, axis=0)
        g1 = 1.0 / denom
        g2 = jnp.exp(m2 - m1) * g1
        me_acc_ref[...] += ex * g1[None, :]

        SB = U_ref.shape[0]
        M12 = jnp.concatenate([M1, M2], axis=0).astype(jnp.bfloat16)
        off = carry_ref[...]
        parts = []
        for s in range(B // SB):
            Cs = lax.dot_general(M12[:, s * SB:(s + 1) * SB], U_ref[...],
                                 dimension_numbers=(((1,), (0,)), ((), ())),
                                 preferred_element_type=jnp.float32) + off
            parts.append(Cs)
            off = Cs[:, SB - 1:SB]
        CC = jnp.concatenate(parts, axis=1)
        rank1 = jnp.sum(jnp.where(M1, CC[:E], 0.0), axis=0) - 1.0
        rank2 = jnp.sum(jnp.where(M2, CC[E:], 0.0), axis=0) - 1.0
        carry_ref[...] = off

        loc1 = jnp.where(rank1 < cap, rank1, 0.0)

        g1_ref[...] = g1
        g2_ref[...] = g2
        e1_ref[...] = e1
        e2_ref[...] = e2
        loc1_ref[...] = loc1.astype(jnp.int32)
        r2_sc[pl.ds(i, 1), :] = rank2.astype(jnp.int32).reshape(1, B)
        e2_sc[pl.ds(i, 1), :] = e2.reshape(1, B)

        @pl.when(i == NB - 1)
        def _fin():
            cnt1 = carry_ref[...][:E]
            cnt1_ref[...] = cnt1
            me = jnp.sum(me_acc_ref[...], axis=1, keepdims=True)
            ce = jnp.minimum(cnt1, float(cap))
            laux_ref[...] = jnp.sum(me * ce, axis=0, keepdims=True) * scale

    @pl.when(i >= NB)
    def _phase2():
        j = i - NB
        r2 = r2_sc[pl.ds(j, 1), :].reshape(B)
        e2b = e2_sc[pl.ds(j, 1), :]
        cnt1 = carry_ref[...][:E]
        row = lax.broadcasted_iota(jnp.int32, (E, B), 0)
        oh = row == e2b
        cntg = jnp.sum(jnp.where(oh, cnt1, 0.0), axis=0)
        loc2 = r2 + cntg.astype(jnp.int32)
        loc2_ref[...] = jnp.where(loc2 < cap, loc2, 0)


def kernel(input, wg):
    N, D = input.shape
    E = wg.shape[0]
    B = 4096
    NB = N // B
    cap = 2 * ((N + E - 1) // E)
    scale = float(E) / (float(N) * float(N))

    SB = 256
    U = jnp.triu(jnp.ones((SB, SB), jnp.bfloat16))

    outs = pl.pallas_call(
        functools.partial(_body, cap, B, E, NB, scale),
        grid=(2 * NB,),
        in_specs=[
            pl.BlockSpec((B, D), lambda i: (jnp.minimum(i, NB - 1), 0)),
            pl.BlockSpec((E, D), lambda i: (0, 0)),
            pl.BlockSpec((SB, SB), lambda i: (0, 0)),
        ],
        out_specs=[
            pl.BlockSpec((B,), lambda i: (jnp.minimum(i, NB - 1),)),
            pl.BlockSpec((B,), lambda i: (jnp.minimum(i, NB - 1),)),
            pl.BlockSpec((B,), lambda i: (jnp.minimum(i, NB - 1),)),
            pl.BlockSpec((B,), lambda i: (jnp.minimum(i, NB - 1),)),
            pl.BlockSpec((B,), lambda i: (jnp.minimum(i, NB - 1),)),
            pl.BlockSpec((B,), lambda i: (jnp.maximum(i - NB, 0),)),
            pl.BlockSpec((E, 1), lambda i: (0, 0)),
            pl.BlockSpec((1, 1), lambda i: (0, 0)),
        ],
        out_shape=[
            jax.ShapeDtypeStruct((N,), jnp.float32),
            jax.ShapeDtypeStruct((N,), jnp.float32),
            jax.ShapeDtypeStruct((N,), jnp.int32),
            jax.ShapeDtypeStruct((N,), jnp.int32),
            jax.ShapeDtypeStruct((N,), jnp.int32),
            jax.ShapeDtypeStruct((N,), jnp.int32),
            jax.ShapeDtypeStruct((E, 1), jnp.float32),
            jax.ShapeDtypeStruct((1, 1), jnp.float32),
        ],
        scratch_shapes=[
            pltpu.VMEM((2 * E, 1), jnp.float32),
            pltpu.VMEM((E, B), jnp.float32),
            pltpu.VMEM((NB, B), jnp.int32),
            pltpu.VMEM((NB, B), jnp.int32),
        ],
        interpret=_INTERPRET,
    )(input, wg, U)
    g1, g2, e1, e2, loc1, loc2, cnt1, laux = outs

    return (laux[0, 0], g1, g2, e1, e2, loc1, loc2)


# speedup vs baseline: 7.8865x; 1.0022x over previous
import functools

import jax
import jax.numpy as jnp
from jax import lax
from jax.experimental import pallas as pl
from jax.experimental.pallas import tpu as pltpu


def _body(cap, B, E, NB, scale, x_ref, wg_ref, U_ref,
          g1_ref, g2_ref, e1_ref, e2_ref, loc1_ref, loc2_ref,
          cnt1_ref, laux_ref, carry_ref, me_acc_ref, r2_sc, e2_sc):
    i = pl.program_id(0)

    @pl.when(i == 0)
    def _init():
        carry_ref[...] = jnp.zeros_like(carry_ref)
        me_acc_ref[...] = jnp.zeros_like(me_acc_ref)

    @pl.when(i < NB)
    def _phase1():
        logitsT = lax.dot_general(
            wg_ref[...], x_ref[...],
            dimension_numbers=(((1,), (1,)), ((), ())),
            preferred_element_type=jnp.float32)

        row = lax.broadcasted_iota(jnp.int32, (E, B), 0)
        m1 = jnp.max(logitsT, axis=0)
        is1 = logitsT == m1[None, :]
        e1 = jnp.min(jnp.where(is1, row, E), axis=0)
        M1 = row == e1[None, :]
        masked = jnp.where(M1, -jnp.inf, logitsT)
        m2 = jnp.max(masked, axis=0)
        is2 = masked == m2[None, :]
        e2 = jnp.min(jnp.where(is2, row, E), axis=0)
        M2 = row == e2[None, :]

        ex = jnp.exp(logitsT - m1[None, :])
        denom = jnp.sum(ex, axis=0)
        g1 = 1.0 / denom
        g2 = jnp.exp(m2 - m1) * g1
        me_acc_ref[...] += ex * g1[None, :]

        SB = U_ref.shape[0]
        M12 = jnp.concatenate([M1, M2], axis=0).astype(jnp.bfloat16)
        off = carry_ref[...]
        parts = []
        for s in range(B // SB):
            Cs = lax.dot_general(M12[:, s * SB:(s + 1) * SB], U_ref[...],
                                 dimension_numbers=(((1,), (0,)), ((), ())),
                                 preferred_element_type=jnp.float32) + off
            parts.append(Cs)
            off = Cs[:, SB - 1:SB]
        CC = jnp.concatenate(parts, axis=1)
        rank1 = jnp.sum(jnp.where(M1, CC[:E], 0.0), axis=0) - 1.0
        rank2 = jnp.sum(jnp.where(M2, CC[E:], 0.0), axis=0) - 1.0
        carry_ref[...] = off

        loc1 = jnp.where(rank1 < cap, rank1, 0.0)

        g1_ref[...] = g1
        g2_ref[...] = g2
        e1_ref[...] = e1
        e2_ref[...] = e2
        loc1_ref[...] = loc1.astype(jnp.int32)
        r2_sc[pl.ds(i, 1), :] = rank2.astype(jnp.int32).reshape(1, B)
        e2_sc[pl.ds(i, 1), :] = e2.reshape(1, B)

        @pl.when(i == NB - 1)
        def _fin():
            cnt1 = carry_ref[...][:E]
            cnt1_ref[...] = cnt1
            me = jnp.sum(me_acc_ref[...], axis=1, keepdims=True)
            ce = jnp.minimum(cnt1, float(cap))
            laux_ref[...] = jnp.sum(me * ce, axis=0, keepdims=True) * scale

    @pl.when(i >= NB)
    def _phase2():
        j = i - NB
        r2 = r2_sc[pl.ds(j, 1), :].reshape(B)
        e2b = e2_sc[pl.ds(j, 1), :]
        cnt1 = carry_ref[...][:E]
        row = lax.broadcasted_iota(jnp.int32, (E, B), 0)
        oh = row == e2b
        cntg = jnp.sum(jnp.where(oh, cnt1, 0.0), axis=0)
        loc2 = r2 + cntg.astype(jnp.int32)
        loc2_ref[...] = jnp.where(loc2 < cap, loc2, 0)


def kernel(input, wg):
    N, D = input.shape
    E = wg.shape[0]
    B = 4096
    NB = N // B
    cap = 2 * ((N + E - 1) // E)
    scale = float(E) / (float(N) * float(N))

    SB = 256
    U = jnp.triu(jnp.ones((SB, SB), jnp.bfloat16))

    outs = pl.pallas_call(
        functools.partial(_body, cap, B, E, NB, scale),
        grid=(2 * NB,),
        in_specs=[
            pl.BlockSpec((B, D), lambda i: (jnp.minimum(i, NB - 1), 0)),
            pl.BlockSpec((E, D), lambda i: (0, 0)),
            pl.BlockSpec((SB, SB), lambda i: (0, 0)),
        ],
        out_specs=[
            pl.BlockSpec((B,), lambda i: (jnp.minimum(i, NB - 1),)),
            pl.BlockSpec((B,), lambda i: (jnp.minimum(i, NB - 1),)),
            pl.BlockSpec((B,), lambda i: (jnp.minimum(i, NB - 1),)),
            pl.BlockSpec((B,), lambda i: (jnp.minimum(i, NB - 1),)),
            pl.BlockSpec((B,), lambda i: (jnp.minimum(i, NB - 1),)),
            pl.BlockSpec((B,), lambda i: (jnp.maximum(i - NB, 0),)),
            pl.BlockSpec((E, 1), lambda i: (0, 0)),
            pl.BlockSpec((1, 1), lambda i: (0, 0)),
        ],
        out_shape=[
            jax.ShapeDtypeStruct((N,), jnp.float32),
            jax.ShapeDtypeStruct((N,), jnp.float32),
            jax.ShapeDtypeStruct((N,), jnp.int32),
            jax.ShapeDtypeStruct((N,), jnp.int32),
            jax.ShapeDtypeStruct((N,), jnp.int32),
            jax.ShapeDtypeStruct((N,), jnp.int32),
            jax.ShapeDtypeStruct((E, 1), jnp.float32),
            jax.ShapeDtypeStruct((1, 1), jnp.float32),
        ],
        scratch_shapes=[
            pltpu.VMEM((2 * E, 1), jnp.float32),
            pltpu.VMEM((E, B), jnp.float32),
            pltpu.VMEM((NB, B), jnp.int32),
            pltpu.VMEM((NB, B), jnp.int32),
        ],
    )(input, wg, U)
    g1, g2, e1, e2, loc1, loc2, cnt1, laux = outs

    return (laux[0, 0], g1, g2, e1, e2, loc1, loc2)
